# baseline (speedup 1.0000x reference)
import functools

import jax
import jax.numpy as jnp
from jax import lax
from jax.experimental import pallas as pl
from jax.experimental.pallas import tpu as pltpu
from jax.experimental.pallas import tpu_sc as plsc

B, N, C = 8, 20000, 80
LANES = 16
CHUNK = 10 * LANES


def _logits_kernel(logits_ref, cent_ref, logits_out_ref):
    logits_out_ref[...] = jnp.sqrt(logits_ref[...] * cent_ref[...])


def _boxes_sc_kernel(rt_hbm, pt_hbm, out_hbm, r_v, p_v, o_v):
    c = lax.axis_index("c")
    s = lax.axis_index("s")
    w = s * 2 + c
    b = w // 4
    k = w % 4
    pltpu.sync_copy(rt_hbm.at[b, k], r_v)
    pltpu.sync_copy(pt_hbm.at[k % 2], p_v)
    sign = jnp.where(k >= 2, 1.0, -1.0).astype(jnp.float32)
    sv = lax.broadcast_in_dim(sign, (LANES,), ())

    def body(i, carry):
        base = i * CHUNK
        for j in range(CHUNK // LANES):
            sl = pl.ds(base + j * LANES, LANES)
            o_v[sl] = jnp.clip(p_v[sl] + sv * r_v[sl], 0.0, 1.0)
        return carry

    lax.fori_loop(0, N // CHUNK, body, 0)
    pltpu.sync_copy(o_v, out_hbm.at[b, k])


def _boxes_sc(rt, pt):
    mesh = plsc.VectorSubcoreMesh(core_axis_name="c", subcore_axis_name="s")
    kern = functools.partial(
        pl.kernel,
        mesh=mesh,
        out_type=jax.ShapeDtypeStruct((B, 4, N), jnp.float32),
        scratch_types=[
            pltpu.VMEM((N,), jnp.float32),
            pltpu.VMEM((N,), jnp.float32),
            pltpu.VMEM((N,), jnp.float32),
        ],
    )(_boxes_sc_kernel)
    return kern(rt, pt)


def kernel(logits, regress, points, centerness):
    lt = jnp.transpose(logits, (0, 2, 1))
    rt = jnp.transpose(regress, (0, 2, 1))
    pt = jnp.transpose(points, (1, 0))
    ct = jnp.transpose(centerness, (0, 2, 1))

    boxes_t = _boxes_sc(rt, pt)

    logits_t = pl.pallas_call(
        _logits_kernel,
        grid=(B,),
        in_specs=[
            pl.BlockSpec((1, C, N), lambda b: (b, 0, 0)),
            pl.BlockSpec((1, 1, N), lambda b: (b, 0, 0)),
        ],
        out_specs=pl.BlockSpec((1, C, N), lambda b: (b, 0, 0)),
        out_shape=jax.ShapeDtypeStruct((B, C, N), jnp.float32),
        compiler_params=pltpu.CompilerParams(
            dimension_semantics=("parallel",),
        ),
    )(lt, ct)
    return (jnp.transpose(logits_t, (0, 2, 1)), jnp.transpose(boxes_t, (0, 2, 1)))


# speedup vs baseline: 1.3978x; 1.3978x over previous
import jax
import jax.numpy as jnp
from jax.experimental import pallas as pl
from jax.experimental.pallas import tpu as pltpu

B, N, C = 8, 20000, 80
BSTEP = 2


def _fused_kernel(logits_ref, cent_ref, regress_ref, pts_ref,
                  logits_out_ref, boxes_out_ref):
    l = logits_ref[...]
    c = cent_ref[...]
    logits_out_ref[...] = jnp.sqrt(l * c)

    r = regress_ref[...]
    px = pts_ref[0:1, :][None]
    py = pts_ref[1:2, :][None]
    row = jax.lax.broadcasted_iota(jnp.int32, r.shape, 1)
    sign = jnp.where(row >= 2, 1.0, -1.0).astype(jnp.float32)
    pts4 = jnp.where(row % 2 == 0, px, py)
    boxes_out_ref[...] = jnp.clip(pts4 + sign * r, 0.0, 1.0)


def kernel(logits, regress, points, centerness):
    lt = jnp.transpose(logits, (0, 2, 1))
    rt = jnp.transpose(regress, (0, 2, 1))
    pt = jnp.transpose(points, (1, 0))
    ct = jnp.transpose(centerness, (0, 2, 1))

    out = pl.pallas_call(
        _fused_kernel,
        grid=(B // BSTEP,),
        in_specs=[
            pl.BlockSpec((BSTEP, C, N), lambda b: (b, 0, 0)),
            pl.BlockSpec((BSTEP, 1, N), lambda b: (b, 0, 0)),
            pl.BlockSpec((BSTEP, 4, N), lambda b: (b, 0, 0)),
            pl.BlockSpec((2, N), lambda b: (0, 0)),
        ],
        out_specs=[
            pl.BlockSpec((BSTEP, C, N), lambda b: (b, 0, 0)),
            pl.BlockSpec((BSTEP, 4, N), lambda b: (b, 0, 0)),
        ],
        out_shape=[
            jax.ShapeDtypeStruct((B, C, N), jnp.float32),
            jax.ShapeDtypeStruct((B, 4, N), jnp.float32),
        ],
        compiler_params=pltpu.CompilerParams(
            dimension_semantics=("parallel",),
        ),
    )(lt, ct, rt, pt)
    return (jnp.transpose(out[0], (0, 2, 1)), jnp.transpose(out[1], (0, 2, 1)))


# speedup vs baseline: 1.4311x; 1.0238x over previous
import jax
import jax.numpy as jnp
from jax.experimental import pallas as pl
from jax.experimental.pallas import tpu as pltpu

B, N, C = 8, 20000, 80
BSTEP = 1


def _fused_kernel(logits_ref, cent_ref, regress_ref, pts_ref,
                  logits_out_ref, boxes_out_ref):
    l = logits_ref[...]
    c = cent_ref[...]
    logits_out_ref[...] = jnp.sqrt(l * c)

    r = regress_ref[...]
    px = pts_ref[0:1, :][None]
    py = pts_ref[1:2, :][None]
    row = jax.lax.broadcasted_iota(jnp.int32, r.shape, 1)
    sign = jnp.where(row >= 2, 1.0, -1.0).astype(jnp.float32)
    pts4 = jnp.where(row % 2 == 0, px, py)
    boxes_out_ref[...] = jnp.clip(pts4 + sign * r, 0.0, 1.0)


def kernel(logits, regress, points, centerness):
    lt = jnp.transpose(logits, (0, 2, 1))
    rt = jnp.transpose(regress, (0, 2, 1))
    pt = jnp.transpose(points, (1, 0))
    ct = jnp.transpose(centerness, (0, 2, 1))

    out = pl.pallas_call(
        _fused_kernel,
        grid=(B // BSTEP,),
        in_specs=[
            pl.BlockSpec((BSTEP, C, N), lambda b: (b, 0, 0)),
            pl.BlockSpec((BSTEP, 1, N), lambda b: (b, 0, 0)),
            pl.BlockSpec((BSTEP, 4, N), lambda b: (b, 0, 0)),
            pl.BlockSpec((2, N), lambda b: (0, 0)),
        ],
        out_specs=[
            pl.BlockSpec((BSTEP, C, N), lambda b: (b, 0, 0)),
            pl.BlockSpec((BSTEP, 4, N), lambda b: (b, 0, 0)),
        ],
        out_shape=[
            jax.ShapeDtypeStruct((B, C, N), jnp.float32),
            jax.ShapeDtypeStruct((B, 4, N), jnp.float32),
        ],
        compiler_params=pltpu.CompilerParams(
            dimension_semantics=("parallel",),
        ),
    )(lt, ct, rt, pt)
    return (jnp.transpose(out[0], (0, 2, 1)), jnp.transpose(out[1], (0, 2, 1)))


# speedup vs baseline: 1.4967x; 1.0458x over previous
import jax
import jax.numpy as jnp
from jax.experimental import pallas as pl
from jax.experimental.pallas import tpu as pltpu

B, N, C = 8, 20000, 80
CSPLIT = 1
BC = C // CSPLIT


def _fused_kernel(logits_ref, cent_ref, regress_ref, pts_ref,
                  logits_out_ref, boxes_out_ref):
    l = logits_ref[...]
    c = cent_ref[...]
    logits_out_ref[...] = jnp.sqrt(l * c)

    @pl.when(pl.program_id(1) == 0)
    def _():
        r = regress_ref[...]
        px = pts_ref[0:1, :][None]
        py = pts_ref[1:2, :][None]
        row = jax.lax.broadcasted_iota(jnp.int32, r.shape, 1)
        sign = jnp.where(row >= 2, 1.0, -1.0).astype(jnp.float32)
        pts4 = jnp.where(row % 2 == 0, px, py)
        boxes_out_ref[...] = jnp.clip(pts4 + sign * r, 0.0, 1.0)


def kernel(logits, regress, points, centerness):
    lt = jnp.transpose(logits, (0, 2, 1))
    rt = jnp.transpose(regress, (0, 2, 1))
    pt = jnp.transpose(points, (1, 0))
    ct = jnp.transpose(centerness, (0, 2, 1))

    out = pl.pallas_call(
        _fused_kernel,
        grid=(B, CSPLIT),
        in_specs=[
            pl.BlockSpec((1, BC, N), lambda b, j: (b, j, 0)),
            pl.BlockSpec((1, 1, N), lambda b, j: (b, 0, 0)),
            pl.BlockSpec((1, 4, N), lambda b, j: (b, 0, 0)),
            pl.BlockSpec((2, N), lambda b, j: (0, 0)),
        ],
        out_specs=[
            pl.BlockSpec((1, BC, N), lambda b, j: (b, j, 0)),
            pl.BlockSpec((1, 4, N), lambda b, j: (b, 0, 0)),
        ],
        out_shape=[
            jax.ShapeDtypeStruct((B, C, N), jnp.float32),
            jax.ShapeDtypeStruct((B, 4, N), jnp.float32),
        ],
        compiler_params=pltpu.CompilerParams(
            dimension_semantics=("parallel", "arbitrary"),
        ),
    )(lt, ct, rt, pt)
    return (jnp.transpose(out[0], (0, 2, 1)), jnp.transpose(out[1], (0, 2, 1)))
